# speedup vs baseline: 1.2632x; 1.2632x over previous
import functools

import jax
import jax.numpy as jnp
from jax import lax
from jax.experimental import pallas as pl
from jax.experimental.pallas import tpu as pltpu
from jax.experimental.pallas import tpu_sc as plsc

B = 16384
F = 26
V = 100000
H = 128
L = 64

NW = 32
ROWS_PER_W = B // NW
RSUB = 4
GSZ = RSUB * F
NSUB = ROWS_PER_W // RSUB
NLANE = 16


def _gather_sum_body(idx_hbm, tables_hbm, out_hbm, idx_v, buf0, buf1, out_v,
                     sem0, sem1):
    c = lax.axis_index("c")
    s = lax.axis_index("s")
    wid = s * 2 + c

    pltpu.sync_copy(idx_hbm.at[pl.ds(wid * NSUB, NSUB)], idx_v)

    bufs = (buf0, buf1)
    sems = (sem0, sem1)

    pltpu.async_copy(tables_hbm.at[idx_v.at[0]], buf0, sem0)

    def accum(g, buf):
        def row_body(r, carry):
            orow = g * RSUB + r
            for j in range(H // NLANE):
                sl = pl.ds(j * NLANE, NLANE)
                vals = [buf[r * F + f, sl] for f in range(F)]
                while len(vals) > 1:
                    nxt = [vals[i] + vals[i + 1]
                           for i in range(0, len(vals) - 1, 2)]
                    if len(vals) % 2:
                        nxt.append(vals[-1])
                    vals = nxt
                out_v[orow, sl] = vals[0]
            return carry
        lax.fori_loop(0, RSUB, row_body, 0)

    def outer(i, carry):
        for b in range(2):
            g = i * 2 + b
            buf, sem = bufs[b], sems[b]
            nbuf, nsem = bufs[1 - b], sems[1 - b]

            @pl.when(g + 1 < NSUB)
            def _issue():
                pltpu.async_copy(tables_hbm.at[idx_v.at[g + 1]], nbuf, nsem)

            pltpu.make_async_copy(tables_hbm.at[idx_v.at[g]], buf, sem).wait()
            accum(g, buf)
        return carry

    lax.fori_loop(0, NSUB // 2, outer, 0)

    pltpu.sync_copy(out_v, out_hbm.at[pl.ds(wid * ROWS_PER_W, ROWS_PER_W)])


@jax.jit
def _gather_sum(idx2, tables2d):
    mesh = plsc.VectorSubcoreMesh(core_axis_name="c", subcore_axis_name="s")
    return pl.kernel(
        _gather_sum_body,
        out_type=jax.ShapeDtypeStruct((B, H), jnp.float32),
        mesh=mesh,
        scratch_types=[
            pltpu.VMEM((NSUB, GSZ), jnp.int32),
            pltpu.VMEM((GSZ, H), jnp.float32),
            pltpu.VMEM((GSZ, H), jnp.float32),
            pltpu.VMEM((ROWS_PER_W, H), jnp.float32),
            pltpu.SemaphoreType.DMA,
            pltpu.SemaphoreType.DMA,
        ],
    )(idx2, tables2d)


def _mlp_body(h_ref, w1_ref, b1_ref, wo_ref, bo_ref, out_ref):
    h = h_ref[...]
    z = jnp.dot(h, w1_ref[...], preferred_element_type=jnp.float32)
    z = jnp.maximum(z + b1_ref[...], 0.0)
    out_ref[...] = (
        jnp.dot(z, wo_ref[...], preferred_element_type=jnp.float32)
        + bo_ref[...]
    )


@jax.jit
def _mlp(h, W1, b1, Wo, bo):
    TB = 1024
    grid = (B // TB,)
    return pl.pallas_call(
        _mlp_body,
        grid=grid,
        in_specs=[
            pl.BlockSpec((TB, H), lambda i: (i, 0)),
            pl.BlockSpec((H, H), lambda i: (0, 0)),
            pl.BlockSpec((1, H), lambda i: (0, 0)),
            pl.BlockSpec((H, 2 * L), lambda i: (0, 0)),
            pl.BlockSpec((1, 2 * L), lambda i: (0, 0)),
        ],
        out_specs=pl.BlockSpec((TB, 2 * L), lambda i: (i, 0)),
        out_shape=jax.ShapeDtypeStruct((B, 2 * L), jnp.float32),
    )(h, W1, b1, Wo, bo)


def kernel(x, tables, W1, b1, Wmu, bmu, Wvar, bvar):
    tables2d = tables.reshape(F * V, H)
    offs = jnp.arange(F, dtype=jnp.int32) * V
    idx2 = (x.astype(jnp.int32) + offs[None, :]).reshape(B // RSUB, GSZ)
    h = _gather_sum(idx2, tables2d)
    Wo = jnp.concatenate([Wmu, Wvar], axis=1)
    bo = jnp.concatenate([bmu, bvar]).reshape(1, 2 * L)
    out = _mlp(h, W1, b1.reshape(1, H), Wo, bo)
    return out[:, :L], out[:, L:]


# speedup vs baseline: 1.3212x; 1.0459x over previous
import functools

import jax
import jax.numpy as jnp
from jax import lax
from jax.experimental import pallas as pl
from jax.experimental.pallas import tpu as pltpu
from jax.experimental.pallas import tpu_sc as plsc

B = 16384
F = 26
V = 100000
H = 128
L = 64

NW = 32
ROWS_PER_W = B // NW
RSUB = 4
GSZ = RSUB * F
NSUB = ROWS_PER_W // RSUB
NLANE = 16


def _gather_sum_body(idx_hbm, tables_hbm, out_hbm, idx_v, buf0, buf1, out_v,
                     sem0, sem1):
    c = lax.axis_index("c")
    s = lax.axis_index("s")
    wid = s * 2 + c

    pltpu.sync_copy(idx_hbm.at[pl.ds(wid * NSUB, NSUB)], idx_v)

    bufs = (buf0, buf1)
    sems = (sem0, sem1)

    pltpu.async_copy(tables_hbm.at[idx_v.at[0]], buf0, sem0)

    def accum(g, buf):
        def row_body(r, carry):
            orow = g * RSUB + r
            for j in range(H // NLANE):
                sl = pl.ds(j * NLANE, NLANE)
                vals = [buf[r * F + f, sl] for f in range(F)]
                while len(vals) > 1:
                    nxt = [vals[i] + vals[i + 1]
                           for i in range(0, len(vals) - 1, 2)]
                    if len(vals) % 2:
                        nxt.append(vals[-1])
                    vals = nxt
                out_v[orow, sl] = vals[0]
            return carry
        lax.fori_loop(0, RSUB, row_body, 0)

    def outer(i, carry):
        for b in range(2):
            g = i * 2 + b
            buf, sem = bufs[b], sems[b]
            nbuf, nsem = bufs[1 - b], sems[1 - b]

            @pl.when(g + 1 < NSUB)
            def _issue():
                pltpu.async_copy(tables_hbm.at[idx_v.at[g + 1]], nbuf, nsem)

            pltpu.make_async_copy(tables_hbm.at[idx_v.at[g]], buf, sem).wait()
            accum(g, buf)
        return carry

    lax.fori_loop(0, NSUB // 2, outer, 0)

    pltpu.sync_copy(out_v, out_hbm.at[pl.ds(wid * ROWS_PER_W, ROWS_PER_W)])


@jax.jit
def _gather_sum(idx2, tables2d):
    mesh = plsc.VectorSubcoreMesh(core_axis_name="c", subcore_axis_name="s")
    return pl.kernel(
        _gather_sum_body,
        out_type=jax.ShapeDtypeStruct((B, H), jnp.float32),
        mesh=mesh,
        scratch_types=[
            pltpu.VMEM((NSUB, GSZ), jnp.int32),
            pltpu.VMEM((GSZ, H), jnp.float32),
            pltpu.VMEM((GSZ, H), jnp.float32),
            pltpu.VMEM((ROWS_PER_W, H), jnp.float32),
            pltpu.SemaphoreType.DMA,
            pltpu.SemaphoreType.DMA,
        ],
    )(idx2, tables2d)


def _mlp_body(h_ref, w1_ref, b1_ref, wo_ref, bo_ref, mu_ref, lv_ref):
    h = h_ref[...]
    z = jnp.dot(h, w1_ref[...], preferred_element_type=jnp.float32)
    z = jnp.maximum(z + b1_ref[...], 0.0)
    z2 = (
        jnp.dot(z, wo_ref[...], preferred_element_type=jnp.float32)
        + bo_ref[...]
    )
    mu_ref[...] = z2[:, :L]
    lv_ref[...] = z2[:, L:]


@jax.jit
def _mlp(h, W1, b1, Wo, bo):
    TB = 1024
    nb = h.shape[0]
    grid = (nb // TB,)
    return pl.pallas_call(
        _mlp_body,
        grid=grid,
        in_specs=[
            pl.BlockSpec((TB, H), lambda i: (i, 0)),
            pl.BlockSpec((H, H), lambda i: (0, 0)),
            pl.BlockSpec((1, H), lambda i: (0, 0)),
            pl.BlockSpec((H, 2 * L), lambda i: (0, 0)),
            pl.BlockSpec((1, 2 * L), lambda i: (0, 0)),
        ],
        out_specs=[
            pl.BlockSpec((TB, L), lambda i: (i, 0)),
            pl.BlockSpec((TB, L), lambda i: (i, 0)),
        ],
        out_shape=[
            jax.ShapeDtypeStruct((nb, L), jnp.float32),
            jax.ShapeDtypeStruct((nb, L), jnp.float32),
        ],
    )(h, W1, b1, Wo, bo)


def kernel(x, tables, W1, b1, Wmu, bmu, Wvar, bvar):
    tables2d = tables.reshape(F * V, H)
    offs = jnp.arange(F, dtype=jnp.int32) * V
    idx2 = (x.astype(jnp.int32) + offs[None, :]).reshape(B // RSUB, GSZ)
    h = _gather_sum(idx2, tables2d)
    Wo = jnp.concatenate([Wmu, Wvar], axis=1)
    bo = jnp.concatenate([bmu, bvar]).reshape(1, 2 * L)
    mu, lv = _mlp(h, W1, b1.reshape(1, H), Wo, bo)
    return mu, lv
